# initial kernel scaffold (unmeasured)
import jax
import jax.numpy as jnp
from jax import lax
from jax.experimental import pallas as pl
from jax.experimental.pallas import tpu as pltpu

N_DEV = 8
N_HALVES = 2


def kernel(x, w_mat):
    m, k_shard = x.shape
    _, n = w_mat.shape
    m_chunk = m // N_DEV
    nh = n // N_HALVES

    def body(x_ref, w_ref, out_ref, send_ref, recv_ref,
             send_sem, recv_sem, credit_sem):
        d = lax.axis_index("i")
        left = (d - 1) % N_DEV
        right = (d + 1) % N_DEV

        barrier = pltpu.get_barrier_semaphore()
        for nbr in (left, right):
            pl.semaphore_signal(barrier, inc=1, device_id=(nbr,),
                                device_id_type=pl.DeviceIdType.MESH)
        pl.semaphore_wait(barrier, 2)

        def silu(v):
            return v * jax.nn.sigmoid(v)

        prev_send = [None]

        for h in range(N_HALVES):
            col0 = h * nh
            first_half = h == 0
            last_half = h == N_HALVES - 1
            for s in range(N_DEV):
                c = (d - s - 1) % N_DEV
                xc = x_ref[pl.ds(c * m_chunk, m_chunk), :]
                local = jnp.dot(xc, w_ref[:, col0:col0 + nh],
                                preferred_element_type=jnp.float32)
                if s == 0:
                    acc = local
                else:
                    recv = pltpu.make_async_remote_copy(
                        src_ref=recv_ref, dst_ref=recv_ref,
                        send_sem=send_sem, recv_sem=recv_sem,
                        device_id=(left,),
                        device_id_type=pl.DeviceIdType.MESH,
                    )
                    recv.wait_recv()
                    acc = local + recv_ref[:, :].astype(jnp.float32)
                if s < N_DEV - 1:
                    if prev_send[0] is not None:
                        prev_send[0].wait_send()
                    send_ref[:, :] = acc.astype(jnp.bfloat16)
                    if not (first_half and s == 0):
                        pl.semaphore_wait(credit_sem, 1)
                    rdma = pltpu.make_async_remote_copy(
                        src_ref=send_ref, dst_ref=recv_ref,
                        send_sem=send_sem, recv_sem=recv_sem,
                        device_id=(right,),
                        device_id_type=pl.DeviceIdType.MESH,
                    )
                    rdma.start()
                    prev_send[0] = rdma
                    if s >= 1:
                        pl.semaphore_signal(credit_sem, inc=1,
                                            device_id=(left,),
                                            device_id_type=pl.DeviceIdType.MESH)
                else:
                    out_ref[:, col0:col0 + nh] = silu(acc)
                    if not last_half:
                        pl.semaphore_signal(credit_sem, inc=1,
                                            device_id=(left,),
                                            device_id_type=pl.DeviceIdType.MESH)

        prev_send[0].wait_send()

    out_shape = jax.ShapeDtypeStruct((m_chunk, n), jnp.float32)
    return pl.pallas_call(
        body,
        out_shape=out_shape,
        in_specs=[pl.BlockSpec(memory_space=pltpu.VMEM),
                  pl.BlockSpec(memory_space=pltpu.VMEM)],
        out_specs=pl.BlockSpec(memory_space=pltpu.VMEM),
        scratch_shapes=[
            pltpu.VMEM((m_chunk, nh), jnp.bfloat16),
            pltpu.VMEM((m_chunk, nh), jnp.bfloat16),
            pltpu.SemaphoreType.DMA,
            pltpu.SemaphoreType.DMA,
            pltpu.SemaphoreType.REGULAR,
        ],
        compiler_params=pltpu.CompilerParams(collective_id=0),
    )(x, w_mat)


# baseline (device time: 802769 ns/iter reference)
import jax
import jax.numpy as jnp
from jax import lax
from jax.experimental import pallas as pl
from jax.experimental.pallas import tpu as pltpu

N_DEV = 8
N_SPLITS = 4


def kernel(x, w_mat):
    x = x.astype(jnp.bfloat16)
    w_mat = w_mat.astype(jnp.bfloat16)
    m, k_shard = x.shape
    _, n = w_mat.shape
    m_chunk = m // N_DEV
    nh = n // N_SPLITS

    def body(x_ref, w_ref, out_ref, send_ref, recv_ref,
             send_sem, recv_sem, credit_sem):
        d = lax.axis_index("i")
        left = (d - 1) % N_DEV
        right = (d + 1) % N_DEV

        barrier = pltpu.get_barrier_semaphore()
        for nbr in (left, right):
            pl.semaphore_signal(barrier, inc=1, device_id=(nbr,),
                                device_id_type=pl.DeviceIdType.MESH)
        pl.semaphore_wait(barrier, 2)

        def silu(v):
            return v * jax.nn.sigmoid(v)

        prev_send = [None]

        for h in range(N_SPLITS):
            col0 = h * nh
            first_half = h == 0
            last_half = h == N_SPLITS - 1
            for s in range(N_DEV):
                c = (d - s - 1) % N_DEV
                xc = x_ref[pl.ds(c * m_chunk, m_chunk), :]
                local = jnp.dot(xc, w_ref[:, col0:col0 + nh],
                                preferred_element_type=jnp.float32)
                if s == 0:
                    acc = local
                else:
                    recv = pltpu.make_async_remote_copy(
                        src_ref=recv_ref, dst_ref=recv_ref,
                        send_sem=send_sem, recv_sem=recv_sem,
                        device_id=(left,),
                        device_id_type=pl.DeviceIdType.MESH,
                    )
                    recv.wait_recv()
                    acc = local + recv_ref[:, :].astype(jnp.float32)
                if s < N_DEV - 1:
                    if prev_send[0] is not None:
                        prev_send[0].wait_send()
                    send_ref[:, :] = acc.astype(jnp.bfloat16)
                    if s >= 1:
                        pl.semaphore_signal(credit_sem, inc=1,
                                            device_id=(left,),
                                            device_id_type=pl.DeviceIdType.MESH)
                    if not (first_half and s == 0):
                        pl.semaphore_wait(credit_sem, 1)
                    rdma = pltpu.make_async_remote_copy(
                        src_ref=send_ref, dst_ref=recv_ref,
                        send_sem=send_sem, recv_sem=recv_sem,
                        device_id=(right,),
                        device_id_type=pl.DeviceIdType.MESH,
                    )
                    rdma.start()
                    prev_send[0] = rdma
                else:
                    out_ref[:, col0:col0 + nh] = silu(acc)
                    if not last_half:
                        pl.semaphore_signal(credit_sem, inc=1,
                                            device_id=(left,),
                                            device_id_type=pl.DeviceIdType.MESH)

        prev_send[0].wait_send()

    out_shape = jax.ShapeDtypeStruct((m_chunk, n), jnp.float32)
    return pl.pallas_call(
        body,
        out_shape=out_shape,
        in_specs=[pl.BlockSpec(memory_space=pltpu.VMEM),
                  pl.BlockSpec(memory_space=pltpu.VMEM)],
        out_specs=pl.BlockSpec(memory_space=pltpu.VMEM),
        scratch_shapes=[
            pltpu.VMEM((m_chunk, nh), jnp.bfloat16),
            pltpu.VMEM((m_chunk, nh), jnp.bfloat16),
            pltpu.SemaphoreType.DMA,
            pltpu.SemaphoreType.DMA,
            pltpu.SemaphoreType.REGULAR,
        ],
        compiler_params=pltpu.CompilerParams(collective_id=0),
    )(x, w_mat)


# device time: 354907 ns/iter; 2.2619x vs baseline; 2.2619x over previous
import jax
import jax.numpy as jnp
from jax import lax
from jax.experimental import pallas as pl
from jax.experimental.pallas import tpu as pltpu

N_DEV = 8
N_STRIPS = 4
RINGS_PER_DIR = 2


def kernel(x, w_mat):
    x = x.astype(jnp.bfloat16)
    w_mat = w_mat.astype(jnp.bfloat16)
    m, k_shard = x.shape
    _, n = w_mat.shape
    m_chunk = m // N_DEV
    nh = n // N_STRIPS

    def body(x_ref, w_ref, out_ref,
             send_r, recv_r, send_l, recv_l,
             ssem_r, rsem_r, ssem_l, rsem_l,
             credit_r, credit_l):
        d = lax.axis_index("i")
        left = (d - 1) % N_DEV
        right = (d + 1) % N_DEV

        barrier = pltpu.get_barrier_semaphore()
        for nbr in (left, right):
            pl.semaphore_signal(barrier, inc=1, device_id=(nbr,),
                                device_id_type=pl.DeviceIdType.MESH)
        pl.semaphore_wait(barrier, 2)

        def silu(v):
            return v * jax.nn.sigmoid(v)

        prev_send = {}

        for s in range(N_DEV):
            for r in range(RINGS_PER_DIR):
                for dir_name in ("R", "L"):
                    if dir_name == "R":
                        c = (d - s - 1) % N_DEV
                        strip = r
                        sbuf, rbuf = send_r, recv_r
                        ssem, rsem, credit = ssem_r, rsem_r, credit_r
                        dst, upstream = right, left
                    else:
                        c = (d + s + 1) % N_DEV
                        strip = RINGS_PER_DIR + r
                        sbuf, rbuf = send_l, recv_l
                        ssem, rsem, credit = ssem_l, rsem_l, credit_l
                        dst, upstream = left, right
                    col0 = strip * nh
                    local = jnp.dot(
                        x_ref[pl.ds(c * m_chunk, m_chunk), :],
                        w_ref[:, col0:col0 + nh],
                        preferred_element_type=jnp.float32,
                    )
                    if s == 0:
                        acc = local
                    else:
                        recv = pltpu.make_async_remote_copy(
                            src_ref=rbuf.at[r], dst_ref=rbuf.at[r],
                            send_sem=ssem.at[r], recv_sem=rsem.at[r],
                            device_id=(upstream,),
                            device_id_type=pl.DeviceIdType.MESH,
                        )
                        recv.wait_recv()
                        acc = local + rbuf[r].astype(jnp.float32)
                    if s < N_DEV - 1:
                        key = (dir_name, r)
                        if key in prev_send:
                            prev_send[key].wait_send()
                        sbuf[r] = acc.astype(jnp.bfloat16)
                        if s >= 1:
                            pl.semaphore_signal(
                                credit, inc=1, device_id=(upstream,),
                                device_id_type=pl.DeviceIdType.MESH)
                            pl.semaphore_wait(credit, 1)
                        rdma = pltpu.make_async_remote_copy(
                            src_ref=sbuf.at[r], dst_ref=rbuf.at[r],
                            send_sem=ssem.at[r], recv_sem=rsem.at[r],
                            device_id=(dst,),
                            device_id_type=pl.DeviceIdType.MESH,
                        )
                        rdma.start()
                        prev_send[key] = rdma
                    else:
                        out_ref[:, col0:col0 + nh] = silu(acc)

        for rdma in prev_send.values():
            rdma.wait_send()

    out_shape = jax.ShapeDtypeStruct((m_chunk, n), jnp.float32)
    comm = pltpu.VMEM((RINGS_PER_DIR, m_chunk, nh), jnp.bfloat16)
    return pl.pallas_call(
        body,
        out_shape=out_shape,
        in_specs=[pl.BlockSpec(memory_space=pltpu.VMEM),
                  pl.BlockSpec(memory_space=pltpu.VMEM)],
        out_specs=pl.BlockSpec(memory_space=pltpu.VMEM),
        scratch_shapes=[
            comm, comm, comm, comm,
            pltpu.SemaphoreType.DMA((RINGS_PER_DIR,)),
            pltpu.SemaphoreType.DMA((RINGS_PER_DIR,)),
            pltpu.SemaphoreType.DMA((RINGS_PER_DIR,)),
            pltpu.SemaphoreType.DMA((RINGS_PER_DIR,)),
            pltpu.SemaphoreType.REGULAR,
            pltpu.SemaphoreType.REGULAR,
        ],
        compiler_params=pltpu.CompilerParams(
            collective_id=0,
            vmem_limit_bytes=38 * 1024 * 1024,
        ),
    )(x, w_mat)


# device time: 351576 ns/iter; 2.2833x vs baseline; 1.0095x over previous
import jax
import jax.numpy as jnp
from jax import lax
from jax.experimental import pallas as pl
from jax.experimental.pallas import tpu as pltpu

N_DEV = 8
N_STRIPS = 8
RINGS_PER_DIR = 4


def kernel(x, w_mat):
    x = x.astype(jnp.bfloat16)
    w_mat = w_mat.astype(jnp.bfloat16)
    m, k_shard = x.shape
    _, n = w_mat.shape
    m_chunk = m // N_DEV
    nh = n // N_STRIPS

    def body(x_ref, w_ref, out_ref,
             send_r, recv_r, send_l, recv_l,
             ssem_r, rsem_r, ssem_l, rsem_l,
             credit_r, credit_l):
        d = lax.axis_index("i")
        left = (d - 1) % N_DEV
        right = (d + 1) % N_DEV

        barrier = pltpu.get_barrier_semaphore()
        for nbr in (left, right):
            pl.semaphore_signal(barrier, inc=1, device_id=(nbr,),
                                device_id_type=pl.DeviceIdType.MESH)
        pl.semaphore_wait(barrier, 2)

        def silu(v):
            return v * jax.nn.sigmoid(v)

        prev_send = {}

        for s in range(N_DEV):
            for r in range(RINGS_PER_DIR):
                for dir_name in ("R", "L"):
                    if dir_name == "R":
                        c = (d - s - 1) % N_DEV
                        strip = r
                        sbuf, rbuf = send_r, recv_r
                        ssem, rsem, credit = ssem_r, rsem_r, credit_r
                        dst, upstream = right, left
                    else:
                        c = (d + s + 1) % N_DEV
                        strip = RINGS_PER_DIR + r
                        sbuf, rbuf = send_l, recv_l
                        ssem, rsem, credit = ssem_l, rsem_l, credit_l
                        dst, upstream = left, right
                    col0 = strip * nh
                    local = jnp.dot(
                        x_ref[pl.ds(c * m_chunk, m_chunk), :],
                        w_ref[:, col0:col0 + nh],
                        preferred_element_type=jnp.float32,
                    )
                    if s == 0:
                        acc = local
                    else:
                        recv = pltpu.make_async_remote_copy(
                            src_ref=rbuf.at[r], dst_ref=rbuf.at[r],
                            send_sem=ssem.at[r], recv_sem=rsem.at[r],
                            device_id=(upstream,),
                            device_id_type=pl.DeviceIdType.MESH,
                        )
                        recv.wait_recv()
                        acc = local + rbuf[r].astype(jnp.float32)
                    if s < N_DEV - 1:
                        key = (dir_name, r)
                        if key in prev_send:
                            prev_send[key].wait_send()
                        sbuf[r] = acc.astype(jnp.bfloat16)
                        if s >= 1:
                            pl.semaphore_signal(
                                credit, inc=1, device_id=(upstream,),
                                device_id_type=pl.DeviceIdType.MESH)
                            pl.semaphore_wait(credit, 1)
                        rdma = pltpu.make_async_remote_copy(
                            src_ref=sbuf.at[r], dst_ref=rbuf.at[r],
                            send_sem=ssem.at[r], recv_sem=rsem.at[r],
                            device_id=(dst,),
                            device_id_type=pl.DeviceIdType.MESH,
                        )
                        rdma.start()
                        prev_send[key] = rdma
                    else:
                        out_ref[:, col0:col0 + nh] = silu(acc)

        for rdma in prev_send.values():
            rdma.wait_send()

    out_shape = jax.ShapeDtypeStruct((m_chunk, n), jnp.float32)
    comm = pltpu.VMEM((RINGS_PER_DIR, m_chunk, nh), jnp.bfloat16)
    return pl.pallas_call(
        body,
        out_shape=out_shape,
        in_specs=[pl.BlockSpec(memory_space=pltpu.VMEM),
                  pl.BlockSpec(memory_space=pltpu.VMEM)],
        out_specs=pl.BlockSpec(memory_space=pltpu.VMEM),
        scratch_shapes=[
            comm, comm, comm, comm,
            pltpu.SemaphoreType.DMA((RINGS_PER_DIR,)),
            pltpu.SemaphoreType.DMA((RINGS_PER_DIR,)),
            pltpu.SemaphoreType.DMA((RINGS_PER_DIR,)),
            pltpu.SemaphoreType.DMA((RINGS_PER_DIR,)),
            pltpu.SemaphoreType.REGULAR,
            pltpu.SemaphoreType.REGULAR,
        ],
        compiler_params=pltpu.CompilerParams(
            collective_id=0,
            vmem_limit_bytes=38 * 1024 * 1024,
        ),
    )(x, w_mat)


# device time: 347903 ns/iter; 2.3075x vs baseline; 1.0106x over previous
import jax
import jax.numpy as jnp
from jax import lax
from jax.experimental import pallas as pl
from jax.experimental.pallas import tpu as pltpu

N_DEV = 8
N_STRIPS = 8
RINGS_PER_DIR = 4


def kernel(x, w_mat):
    m, k_shard = x.shape
    _, n = w_mat.shape
    m_chunk = m // N_DEV
    nh = n // N_STRIPS

    def body(x_ref, w_ref, out_ref,
             send_r, recv_r, send_l, recv_l, staging,
             ssem_r, rsem_r, ssem_l, rsem_l, out_sems,
             credit_r, credit_l):
        d = lax.axis_index("i")
        left = (d - 1) % N_DEV
        right = (d + 1) % N_DEV

        barrier = pltpu.get_barrier_semaphore()
        for nbr in (left, right):
            pl.semaphore_signal(barrier, inc=1, device_id=(nbr,),
                                device_id_type=pl.DeviceIdType.MESH)
        pl.semaphore_wait(barrier, 2)

        def silu(v):
            return v * jax.nn.sigmoid(v)

        prev_send = {}
        out_copies = [None, None]
        n_finals = [0]

        for s in range(N_DEV):
            for r in range(RINGS_PER_DIR):
                for dir_name in ("R", "L"):
                    if dir_name == "R":
                        c = (d - s - 1) % N_DEV
                        strip = r
                        sbuf, rbuf = send_r, recv_r
                        ssem, rsem, credit = ssem_r, rsem_r, credit_r
                        dst, upstream = right, left
                    else:
                        c = (d + s + 1) % N_DEV
                        strip = RINGS_PER_DIR + r
                        sbuf, rbuf = send_l, recv_l
                        ssem, rsem, credit = ssem_l, rsem_l, credit_l
                        dst, upstream = left, right
                    col0 = strip * nh
                    local = jnp.dot(
                        x_ref[pl.ds(c * m_chunk, m_chunk), :],
                        w_ref[:, col0:col0 + nh],
                        preferred_element_type=jnp.float32,
                    )
                    if s == 0:
                        acc = local
                    else:
                        recv = pltpu.make_async_remote_copy(
                            src_ref=rbuf.at[r], dst_ref=rbuf.at[r],
                            send_sem=ssem.at[r], recv_sem=rsem.at[r],
                            device_id=(upstream,),
                            device_id_type=pl.DeviceIdType.MESH,
                        )
                        recv.wait_recv()
                        acc = local + rbuf[r].astype(jnp.float32)
                    if s < N_DEV - 1:
                        key = (dir_name, r)
                        if key in prev_send:
                            prev_send[key].wait_send()
                        sbuf[r] = acc.astype(jnp.bfloat16)
                        if s >= 1:
                            pl.semaphore_signal(
                                credit, inc=1, device_id=(upstream,),
                                device_id_type=pl.DeviceIdType.MESH)
                            pl.semaphore_wait(credit, 1)
                        rdma = pltpu.make_async_remote_copy(
                            src_ref=sbuf.at[r], dst_ref=rbuf.at[r],
                            send_sem=ssem.at[r], recv_sem=rsem.at[r],
                            device_id=(dst,),
                            device_id_type=pl.DeviceIdType.MESH,
                        )
                        rdma.start()
                        prev_send[key] = rdma
                    else:
                        j = n_finals[0] % 2
                        n_finals[0] += 1
                        if out_copies[j] is not None:
                            out_copies[j].wait()
                        staging[j] = silu(acc)
                        cp = pltpu.make_async_copy(
                            staging.at[j],
                            out_ref.at[:, pl.ds(col0, nh)],
                            out_sems.at[j],
                        )
                        cp.start()
                        out_copies[j] = cp

        for rdma in prev_send.values():
            rdma.wait_send()
        for cp in out_copies:
            cp.wait()

    out_shape = jax.ShapeDtypeStruct((m_chunk, n), jnp.float32)
    comm = pltpu.VMEM((RINGS_PER_DIR, m_chunk, nh), jnp.bfloat16)
    return pl.pallas_call(
        body,
        out_shape=out_shape,
        in_specs=[pl.BlockSpec(memory_space=pltpu.VMEM),
                  pl.BlockSpec(memory_space=pltpu.VMEM)],
        out_specs=pl.BlockSpec(memory_space=pl.ANY),
        scratch_shapes=[
            comm, comm, comm, comm,
            pltpu.VMEM((2, m_chunk, nh), jnp.float32),
            pltpu.SemaphoreType.DMA((RINGS_PER_DIR,)),
            pltpu.SemaphoreType.DMA((RINGS_PER_DIR,)),
            pltpu.SemaphoreType.DMA((RINGS_PER_DIR,)),
            pltpu.SemaphoreType.DMA((RINGS_PER_DIR,)),
            pltpu.SemaphoreType.DMA((2,)),
            pltpu.SemaphoreType.REGULAR,
            pltpu.SemaphoreType.REGULAR,
        ],
        compiler_params=pltpu.CompilerParams(
            collective_id=0,
            vmem_limit_bytes=40 * 1024 * 1024,
        ),
    )(x, w_mat)


# device time: 346653 ns/iter; 2.3158x vs baseline; 1.0036x over previous
import jax
import jax.numpy as jnp
from jax import lax
from jax.experimental import pallas as pl
from jax.experimental.pallas import tpu as pltpu

N_DEV = 8
N_STRIPS = 8
RINGS_PER_DIR = 4


def kernel(x, w_mat):
    m, k_shard = x.shape
    _, n = w_mat.shape
    m_chunk = m // N_DEV
    nh = n // N_STRIPS

    def body(x_ref, w_ref, out_ref,
             send_r, recv_r, send_l, recv_l, staging,
             ssem_r, rsem_r, ssem_l, rsem_l, out_sems,
             credit_r, credit_l):
        d = lax.axis_index("i")
        left = (d - 1) % N_DEV
        right = (d + 1) % N_DEV

        barrier = pltpu.get_barrier_semaphore()
        for nbr in (left, right):
            pl.semaphore_signal(barrier, inc=1, device_id=(nbr,),
                                device_id_type=pl.DeviceIdType.MESH)
        pl.semaphore_wait(barrier, 2)

        def silu(v):
            return v * jax.nn.sigmoid(v)

        prev_send = {}
        out_copies = [None, None]
        n_finals = [0]

        for s in range(N_DEV):
            for r in range(RINGS_PER_DIR):
                for dir_name in ("R", "L"):
                    if dir_name == "R":
                        c = (d - s - 1) % N_DEV
                        strip = r
                        sbuf, rbuf = send_r, recv_r
                        ssem, rsem, credit = ssem_r, rsem_r, credit_r
                        dst, upstream = right, left
                    else:
                        c = (d + s + 1) % N_DEV
                        strip = RINGS_PER_DIR + r
                        sbuf, rbuf = send_l, recv_l
                        ssem, rsem, credit = ssem_l, rsem_l, credit_l
                        dst, upstream = left, right
                    col0 = strip * nh
                    local = jnp.dot(
                        x_ref[pl.ds(c * m_chunk, m_chunk), :],
                        w_ref[:, col0:col0 + nh],
                        preferred_element_type=jnp.float32,
                    ).astype(jnp.bfloat16)
                    if s == 0:
                        acc = local
                    else:
                        recv = pltpu.make_async_remote_copy(
                            src_ref=rbuf.at[r], dst_ref=rbuf.at[r],
                            send_sem=ssem.at[r], recv_sem=rsem.at[r],
                            device_id=(upstream,),
                            device_id_type=pl.DeviceIdType.MESH,
                        )
                        recv.wait_recv()
                        acc = local + rbuf[r]
                    if s < N_DEV - 1:
                        key = (dir_name, r)
                        if key in prev_send:
                            prev_send[key].wait_send()
                        sbuf[r] = acc
                        if s >= 1:
                            pl.semaphore_signal(
                                credit, inc=1, device_id=(upstream,),
                                device_id_type=pl.DeviceIdType.MESH)
                            pl.semaphore_wait(credit, 1)
                        rdma = pltpu.make_async_remote_copy(
                            src_ref=sbuf.at[r], dst_ref=rbuf.at[r],
                            send_sem=ssem.at[r], recv_sem=rsem.at[r],
                            device_id=(dst,),
                            device_id_type=pl.DeviceIdType.MESH,
                        )
                        rdma.start()
                        prev_send[key] = rdma
                    else:
                        j = n_finals[0] % 2
                        n_finals[0] += 1
                        if out_copies[j] is not None:
                            out_copies[j].wait()
                        staging[j] = silu(acc.astype(jnp.float32))
                        cp = pltpu.make_async_copy(
                            staging.at[j],
                            out_ref.at[:, pl.ds(col0, nh)],
                            out_sems.at[j],
                        )
                        cp.start()
                        out_copies[j] = cp

        for rdma in prev_send.values():
            rdma.wait_send()
        for cp in out_copies:
            cp.wait()

    out_shape = jax.ShapeDtypeStruct((m_chunk, n), jnp.float32)
    comm = pltpu.VMEM((RINGS_PER_DIR, m_chunk, nh), jnp.bfloat16)
    return pl.pallas_call(
        body,
        out_shape=out_shape,
        in_specs=[pl.BlockSpec(memory_space=pltpu.VMEM),
                  pl.BlockSpec(memory_space=pltpu.VMEM)],
        out_specs=pl.BlockSpec(memory_space=pl.ANY),
        scratch_shapes=[
            comm, comm, comm, comm,
            pltpu.VMEM((2, m_chunk, nh), jnp.float32),
            pltpu.SemaphoreType.DMA((RINGS_PER_DIR,)),
            pltpu.SemaphoreType.DMA((RINGS_PER_DIR,)),
            pltpu.SemaphoreType.DMA((RINGS_PER_DIR,)),
            pltpu.SemaphoreType.DMA((RINGS_PER_DIR,)),
            pltpu.SemaphoreType.DMA((2,)),
            pltpu.SemaphoreType.REGULAR,
            pltpu.SemaphoreType.REGULAR,
        ],
        compiler_params=pltpu.CompilerParams(
            collective_id=0,
            vmem_limit_bytes=42 * 1024 * 1024,
        ),
    )(x, w_mat)
